# Initial kernel scaffold; baseline (speedup 1.0000x reference)
import jax
import jax.numpy as jnp
from jax.experimental import pallas as pl


def kernel(features, captions, emb_table, W1, b1, W2, b2):
    raise NotImplementedError("write your pallas kernel here")


# speedup vs baseline: 3.6310x; 3.6310x over previous
import functools

import jax
import jax.numpy as jnp
from jax import lax
from jax.experimental import pallas as pl
from jax.experimental.pallas import tpu as pltpu
from jax.experimental.pallas import tpu_sc as plsc

_NUM_CORES = 2
_NUM_SUBCORES = 16
_NUM_WORKERS = _NUM_CORES * _NUM_SUBCORES


_NBUF = 6
_WDELAY = 2
_CHUNK_ROWS = 80


def _gather_part(features, idx_flat, emb_table, emb_lo, emb_rows,
                 with_features):
    B, D = features.shape
    span = emb_rows // _NUM_WORKERS
    bpw = B // _NUM_WORKERS
    base = B if with_features else 0
    nrows = base + emb_rows
    chunks = [_CHUNK_ROWS] * (span // _CHUNK_ROWS)
    if span % _CHUNK_ROWS:
        chunks.append(span % _CHUNK_ROWS)
    offs = [0]
    for c in chunks:
        offs.append(offs[-1] + c)
    nch = len(chunks)

    mesh = plsc.VectorSubcoreMesh(
        core_axis_name="c", subcore_axis_name="s",
        num_cores=_NUM_CORES, num_subcores=_NUM_SUBCORES)

    @functools.partial(
        pl.kernel,
        out_type=jax.ShapeDtypeStruct((nrows, D), jnp.float32),
        mesh=mesh,
        scratch_types=(
            [pltpu.VMEM((span,), jnp.int32),
             pltpu.VMEM((bpw, D), jnp.float32),
             pltpu.VMEM((_NBUF, _CHUNK_ROWS, D), jnp.float32)]
            + [pltpu.SemaphoreType.DMA] * (2 * _NBUF + 1)
        ),
    )
    def sc_kernel(idx_hbm, feat_hbm, table_hbm, x_hbm,
                  idx_v, feat_v, bufs, *sems):
        gsems = sems[:_NBUF]
        wsems = sems[_NBUF:2 * _NBUF]
        fsem = sems[2 * _NBUF]
        wid = lax.axis_index("s") * _NUM_CORES + lax.axis_index("c")
        ibase = pl.multiple_of(emb_lo + wid * span, 8)
        xbase = pl.multiple_of(base + wid * span, 8)

        pltpu.sync_copy(idx_hbm.at[pl.ds(ibase, span)], idx_v)
        fd = None
        if with_features:
            fb = pl.multiple_of(wid * bpw, 8)
            pltpu.sync_copy(feat_hbm.at[pl.ds(fb, bpw)], feat_v)
            fd = pltpu.async_copy(feat_v, x_hbm.at[pl.ds(fb, bpw)], fsem)

        gd = [None] * _NBUF
        wd = [None] * _NBUF

        def start_write(j):
            s = j % _NBUF
            gd[s].wait()
            wd[s] = pltpu.async_copy(
                bufs.at[s, pl.ds(0, chunks[j])],
                x_hbm.at[pl.ds(pl.multiple_of(xbase + offs[j], 8),
                               chunks[j])],
                wsems[s])

        for j in range(nch):
            s = j % _NBUF
            if wd[s] is not None:
                wd[s].wait()
            gd[s] = pltpu.async_copy(
                table_hbm.at[idx_v.at[pl.ds(offs[j], chunks[j])]],
                bufs.at[s, pl.ds(0, chunks[j])], gsems[s])
            if j >= _WDELAY:
                start_write(j - _WDELAY)
        for j in range(nch - _WDELAY, nch):
            start_write(j)
        for d in wd:
            if d is not None:
                d.wait()
        if fd is not None:
            fd.wait()

    return sc_kernel(idx_flat, features, emb_table)


def _mlp_half(x_half, W1, b1, W2, b2, tile, N, block0, y_prev):
    rows, D = x_half.shape
    H = W1.shape[1]
    b1r = b1.reshape(1, H)
    b2r = b2.reshape(1, D)

    def body(x_ref, w1_ref, b1_ref, w2_ref, b2_ref, *rest):
        o_ref = rest[-1]
        x = x_ref[...].astype(jnp.bfloat16)
        h = jnp.dot(x, w1_ref[...].astype(jnp.bfloat16),
                    preferred_element_type=jnp.float32)
        h = jnp.maximum(h + b1_ref[...], 0.0).astype(jnp.bfloat16)
        o = jnp.dot(h, w2_ref[...].astype(jnp.bfloat16),
                    preferred_element_type=jnp.float32)
        o_ref[...] = o + b2_ref[...]

    in_specs = [
        pl.BlockSpec((tile, D), lambda i: (i, 0)),
        pl.BlockSpec((D, H), lambda i: (0, 0)),
        pl.BlockSpec((1, H), lambda i: (0, 0)),
        pl.BlockSpec((H, D), lambda i: (0, 0)),
        pl.BlockSpec((1, D), lambda i: (0, 0)),
    ]
    args = [x_half, W1, b1r, W2, b2r]
    aliases = {}
    if y_prev is not None:
        in_specs.append(pl.BlockSpec(memory_space=pl.ANY))
        args.append(y_prev)
        aliases = {5: 0}
    return pl.pallas_call(
        body,
        grid=(rows // tile,),
        in_specs=in_specs,
        out_specs=pl.BlockSpec((tile, D), lambda i: (i + block0, 0)),
        out_shape=jax.ShapeDtypeStruct((N, D), jnp.float32),
        input_output_aliases=aliases,
    )(*args)


def kernel(features, captions, emb_table, W1, b1, W2, b2):
    B, D = features.shape
    L = captions.shape[1]
    Lp = L + 1
    N = B * Lp
    tile = 3264
    half = N // 2
    steps = half // tile
    idx_flat = captions.astype(jnp.int32).T.reshape(B * L)

    x1 = _gather_part(features, idx_flat, emb_table,
                      emb_lo=0, emb_rows=half - B, with_features=True)
    x2 = _gather_part(features, idx_flat, emb_table,
                      emb_lo=half - B, emb_rows=half, with_features=False)
    y1 = _mlp_half(x1, W1, b1, W2, b2, tile, N, 0, None)
    y = _mlp_half(x2, W1, b1, W2, b2, tile, N, steps, y1)
    return y.reshape(Lp, B, D).transpose(1, 0, 2)


# speedup vs baseline: 3.6546x; 1.0065x over previous
import functools

import jax
import jax.numpy as jnp
from jax import lax
from jax.experimental import pallas as pl
from jax.experimental.pallas import tpu as pltpu
from jax.experimental.pallas import tpu_sc as plsc

_NUM_CORES = 2
_NUM_SUBCORES = 16
_NUM_WORKERS = _NUM_CORES * _NUM_SUBCORES


_NBUF = 8
_WDELAY = 3
_CHUNK_ROWS = 80


def _gather_part(features, idx_flat, emb_table, emb_lo, emb_rows,
                 with_features):
    B, D = features.shape
    span = emb_rows // _NUM_WORKERS
    bpw = B // _NUM_WORKERS
    base = B if with_features else 0
    nrows = base + emb_rows
    chunks = [_CHUNK_ROWS] * (span // _CHUNK_ROWS)
    if span % _CHUNK_ROWS:
        chunks.append(span % _CHUNK_ROWS)
    offs = [0]
    for c in chunks:
        offs.append(offs[-1] + c)
    nch = len(chunks)

    mesh = plsc.VectorSubcoreMesh(
        core_axis_name="c", subcore_axis_name="s",
        num_cores=_NUM_CORES, num_subcores=_NUM_SUBCORES)

    @functools.partial(
        pl.kernel,
        out_type=jax.ShapeDtypeStruct((nrows, D), jnp.float32),
        mesh=mesh,
        scratch_types=(
            [pltpu.VMEM((span,), jnp.int32),
             pltpu.VMEM((bpw, D), jnp.float32),
             pltpu.VMEM((_NBUF, _CHUNK_ROWS, D), jnp.float32)]
            + [pltpu.SemaphoreType.DMA] * (2 * _NBUF + 1)
        ),
    )
    def sc_kernel(idx_hbm, feat_hbm, table_hbm, x_hbm,
                  idx_v, feat_v, bufs, *sems):
        gsems = sems[:_NBUF]
        wsems = sems[_NBUF:2 * _NBUF]
        fsem = sems[2 * _NBUF]
        wid = lax.axis_index("s") * _NUM_CORES + lax.axis_index("c")
        ibase = pl.multiple_of(emb_lo + wid * span, 8)
        xbase = pl.multiple_of(base + wid * span, 8)

        pltpu.sync_copy(idx_hbm.at[pl.ds(ibase, span)], idx_v)
        fd = None
        if with_features:
            fb = pl.multiple_of(wid * bpw, 8)
            pltpu.sync_copy(feat_hbm.at[pl.ds(fb, bpw)], feat_v)
            fd = pltpu.async_copy(feat_v, x_hbm.at[pl.ds(fb, bpw)], fsem)

        gd = [None] * _NBUF
        wd = [None] * _NBUF

        def start_write(j):
            s = j % _NBUF
            gd[s].wait()
            wd[s] = pltpu.async_copy(
                bufs.at[s, pl.ds(0, chunks[j])],
                x_hbm.at[pl.ds(pl.multiple_of(xbase + offs[j], 8),
                               chunks[j])],
                wsems[s])

        for j in range(nch):
            s = j % _NBUF
            if wd[s] is not None:
                wd[s].wait()
            gd[s] = pltpu.async_copy(
                table_hbm.at[idx_v.at[pl.ds(offs[j], chunks[j])]],
                bufs.at[s, pl.ds(0, chunks[j])], gsems[s])
            if j >= _WDELAY:
                start_write(j - _WDELAY)
        for j in range(nch - _WDELAY, nch):
            start_write(j)
        for d in wd:
            if d is not None:
                d.wait()
        if fd is not None:
            fd.wait()

    return sc_kernel(idx_flat, features, emb_table)


def _mlp_half(x_half, W1, b1, W2, b2, tile, N, block0, y_prev):
    rows, D = x_half.shape
    H = W1.shape[1]
    b1r = b1.reshape(1, H)
    b2r = b2.reshape(1, D)

    def body(x_ref, w1_ref, b1_ref, w2_ref, b2_ref, *rest):
        o_ref = rest[-1]
        x = x_ref[...].astype(jnp.bfloat16)
        h = jnp.dot(x, w1_ref[...].astype(jnp.bfloat16),
                    preferred_element_type=jnp.float32)
        h = jnp.maximum(h + b1_ref[...], 0.0).astype(jnp.bfloat16)
        o = jnp.dot(h, w2_ref[...].astype(jnp.bfloat16),
                    preferred_element_type=jnp.float32)
        o_ref[...] = o + b2_ref[...]

    in_specs = [
        pl.BlockSpec((tile, D), lambda i: (i, 0)),
        pl.BlockSpec((D, H), lambda i: (0, 0)),
        pl.BlockSpec((1, H), lambda i: (0, 0)),
        pl.BlockSpec((H, D), lambda i: (0, 0)),
        pl.BlockSpec((1, D), lambda i: (0, 0)),
    ]
    args = [x_half, W1, b1r, W2, b2r]
    aliases = {}
    if y_prev is not None:
        in_specs.append(pl.BlockSpec(memory_space=pl.ANY))
        args.append(y_prev)
        aliases = {5: 0}
    return pl.pallas_call(
        body,
        grid=(rows // tile,),
        in_specs=in_specs,
        out_specs=pl.BlockSpec((tile, D), lambda i: (i + block0, 0)),
        out_shape=jax.ShapeDtypeStruct((N, D), jnp.float32),
        input_output_aliases=aliases,
    )(*args)


def kernel(features, captions, emb_table, W1, b1, W2, b2):
    B, D = features.shape
    L = captions.shape[1]
    Lp = L + 1
    N = B * Lp
    tile = 3264
    half = N // 2
    steps = half // tile
    idx_flat = captions.astype(jnp.int32).T.reshape(B * L)

    x1 = _gather_part(features, idx_flat, emb_table,
                      emb_lo=0, emb_rows=half - B, with_features=True)
    x2 = _gather_part(features, idx_flat, emb_table,
                      emb_lo=half - B, emb_rows=half, with_features=False)
    y1 = _mlp_half(x1, W1, b1, W2, b2, tile, N, 0, None)
    y = _mlp_half(x2, W1, b1, W2, b2, tile, N, steps, y1)
    return y.reshape(Lp, B, D).transpose(1, 0, 2)
